# baseline (speedup 1.0000x reference)
import functools

import jax
import jax.numpy as jnp
from jax import lax
from jax.experimental import pallas as pl
from jax.experimental.pallas import tpu as pltpu
from jax.experimental.pallas import tpu_sc as plsc

EMBED_DIM = 64
PAIR_DIM = 2 * EMBED_DIM
IDX_CHUNK = 128
NW = 32
NSLOT = 2


def _cdiv(a, b):
    return (a + b - 1) // b


def _relayout_table(tabT_hbm, tab2_hbm, n_panels, wid,
                    wbuf, obuf, lsems, osems, rowvecs, colvecs):
    ppw = _cdiv(_cdiv(n_panels, NW), NSLOT) * NSLOT
    p0 = wid * ppw
    dump_panel = n_panels

    def fire_load(b, p):
        pc = jnp.minimum(p, n_panels - 1)
        pltpu.async_copy(
            tabT_hbm.at[:, pl.ds(pl.multiple_of(pc * IDX_CHUNK, IDX_CHUNK),
                                 IDX_CHUNK)],
            wbuf[b], lsems[b])

    def fire_out(b, p):
        row = jnp.where(p < n_panels, p, dump_panel) * 64
        pltpu.async_copy(obuf[b].at[:, pl.ds(0, PAIR_DIM)],
                         tab2_hbm.at[pl.ds(row, 64)], osems[b])

    for b in range(NSLOT):
        fire_load(b, p0 + b)
        fire_out(b, jnp.int32(dump_panel))

    def step(it, carry):
        for b in range(NSLOT):
            p = p0 + it * NSLOT + b
            pltpu.make_async_copy(
                tabT_hbm.at[:, pl.ds(0, IDX_CHUNK)],
                wbuf[b], lsems[b]).wait()
            pltpu.make_async_copy(
                obuf[b].at[:, pl.ds(0, PAIR_DIM)],
                tab2_hbm.at[pl.ds(0, 64)], osems[b]).wait()

            for d in range(EMBED_DIM):
                for g in range(8):
                    val = wbuf[b].at[d][pl.ds(g * 16, 16)]
                    plsc.store_scatter(
                        obuf[b], [rowvecs[g], colvecs[g] + d], val)
            fire_out(b, p)
            fire_load(b, p + NSLOT)
        return carry

    lax.fori_loop(0, ppw // NSLOT, step, 0)
    for b in range(NSLOT):
        pltpu.make_async_copy(
            tabT_hbm.at[:, pl.ds(0, IDX_CHUNK)],
            wbuf[b], lsems[b]).wait()
        pltpu.make_async_copy(
            obuf[b].at[:, pl.ds(0, PAIR_DIM)],
            tab2_hbm.at[pl.ds(0, 64)], osems[b]).wait()


def _make_relayout_kernel(n_user, n_anime):
    u_panels = _cdiv(n_user, IDX_CHUNK)
    a_panels = _cdiv(n_anime, IDX_CHUNK)
    u_rows = u_panels * 64 + 64
    a_rows = a_panels * 64 + 64
    mesh = plsc.VectorSubcoreMesh(core_axis_name="c", subcore_axis_name="s")

    @functools.partial(
        pl.kernel,
        out_type=(
            jax.ShapeDtypeStruct((u_rows, PAIR_DIM), jnp.float32),
            jax.ShapeDtypeStruct((a_rows, PAIR_DIM), jnp.float32),
        ),
        compiler_params=pltpu.CompilerParams(needs_layout_passes=False),
        mesh=mesh,
        scratch_types=[pltpu.VMEM((EMBED_DIM, IDX_CHUNK), jnp.float32)] * NSLOT
        + [pltpu.VMEM((64, PAIR_DIM + 1), jnp.float32)] * NSLOT
        + [pltpu.SemaphoreType.DMA] * (2 * NSLOT),
    )
    def relayout_kernel(utabT_hbm, atabT_hbm, utab2_hbm, atab2_hbm, *scratch):
        wbuf = scratch[:NSLOT]
        obuf = scratch[NSLOT:2 * NSLOT]
        sems = scratch[2 * NSLOT:]
        wid = lax.axis_index("s") * 2 + lax.axis_index("c")
        lsems, osems = sems[:NSLOT], sems[NSLOT:]
        rowvecs = []
        colvecs = []
        for g in range(8):
            lanes = g * 16 + lax.iota(jnp.int32, 16)
            rowvecs.append(lanes // 2)
            colvecs.append((lanes % 2) * EMBED_DIM)
        _relayout_table(utabT_hbm, utab2_hbm, u_panels, wid,
                        wbuf, obuf, lsems, osems, rowvecs, colvecs)
        _relayout_table(atabT_hbm, atab2_hbm, a_panels, wid,
                        wbuf, obuf, lsems, osems, rowvecs, colvecs)

    return relayout_kernel


def _make_gather_kernel(batch):
    chunks_per_worker = batch // (NW * IDX_CHUNK)
    rows_per_worker = chunks_per_worker * IDX_CHUNK
    mesh = plsc.VectorSubcoreMesh(core_axis_name="c", subcore_axis_name="s")

    @functools.partial(
        pl.kernel,
        out_type=(
            jax.ShapeDtypeStruct((batch, PAIR_DIM), jnp.float32),
            jax.ShapeDtypeStruct((batch, PAIR_DIM), jnp.float32),
        ),
        mesh=mesh,
        scratch_types=[
            pltpu.VMEM((batch // (NW * IDX_CHUNK), IDX_CHUNK), jnp.int32),
            pltpu.VMEM((batch // (NW * IDX_CHUNK), IDX_CHUNK), jnp.int32),
            pltpu.VMEM((batch // NW, PAIR_DIM), jnp.float32),
            pltpu.SemaphoreType.DMA,
        ],
    )
    def gather_kernel(uidx_hbm, aidx_hbm, utab2_hbm, atab2_hbm,
                      uout_hbm, aout_hbm,
                      uidx_v, aidx_v, rows_v, sem):
        wid = lax.axis_index("s") * 2 + lax.axis_index("c")
        crow = wid * chunks_per_worker
        base = wid * rows_per_worker
        pltpu.sync_copy(uidx_hbm.at[pl.ds(crow, chunks_per_worker)], uidx_v)
        pltpu.sync_copy(aidx_hbm.at[pl.ds(crow, chunks_per_worker)], aidx_v)
        copies = []
        for j in range(chunks_per_worker):
            copies.append(pltpu.async_copy(
                utab2_hbm.at[uidx_v.at[j]],
                rows_v.at[pl.ds(j * IDX_CHUNK, IDX_CHUNK)], sem))
        for c in copies:
            c.wait()
        pltpu.sync_copy(rows_v, uout_hbm.at[pl.ds(base, rows_per_worker)])
        copies = []
        for j in range(chunks_per_worker):
            copies.append(pltpu.async_copy(
                atab2_hbm.at[aidx_v.at[j]],
                rows_v.at[pl.ds(j * IDX_CHUNK, IDX_CHUNK)], sem))
        for c in copies:
            c.wait()
        pltpu.sync_copy(rows_v, aout_hbm.at[pl.ds(base, rows_per_worker)])

    return gather_kernel


def _mlp_body(upair_ref, apair_ref, uid_ref, aid_ref,
              w1u_ref, w1a_ref, b1_ref, w2_ref, b2_ref, w3_ref, out_ref):
    up = upair_ref[...]
    ap = apair_ref[...]
    usel = (uid_ref[...] & 1) == 1
    asel = (aid_ref[...] & 1) == 1
    u = jnp.where(usel, up[:, EMBED_DIM:], up[:, :EMBED_DIM])
    a = jnp.where(asel, ap[:, EMBED_DIM:], ap[:, :EMBED_DIM])
    h1 = jnp.dot(u, w1u_ref[...], preferred_element_type=jnp.float32)
    h1 = h1 + jnp.dot(a, w1a_ref[...], preferred_element_type=jnp.float32)
    h1 = jnp.maximum(h1 + b1_ref[...], 0.0)
    h2 = jnp.dot(h1, w2_ref[...], preferred_element_type=jnp.float32)
    h2 = jnp.maximum(h2 + b2_ref[...], 0.0)
    out_ref[...] = jnp.sum(h2 * w3_ref[...], axis=1)


def _mlp(upairs, apairs, user_id, anime_id, W1, b1, W2, b2, W3, block_b):
    batch = upairs.shape[0]
    grid = (batch // block_b,)
    full = lambda i: (0, 0)
    out = pl.pallas_call(
        _mlp_body,
        grid=grid,
        in_specs=[
            pl.BlockSpec((block_b, PAIR_DIM), lambda i: (i, 0)),
            pl.BlockSpec((block_b, PAIR_DIM), lambda i: (i, 0)),
            pl.BlockSpec((block_b, 1), lambda i: (i, 0)),
            pl.BlockSpec((block_b, 1), lambda i: (i, 0)),
            pl.BlockSpec((EMBED_DIM, 128), full),
            pl.BlockSpec((EMBED_DIM, 128), full),
            pl.BlockSpec((1, 128), full),
            pl.BlockSpec((128, EMBED_DIM), full),
            pl.BlockSpec((1, EMBED_DIM), full),
            pl.BlockSpec((1, EMBED_DIM), full),
        ],
        out_specs=pl.BlockSpec((block_b,), lambda i: (i,)),
        out_shape=jax.ShapeDtypeStruct((batch,), jnp.float32),
    )(upairs, apairs, user_id[:, None], anime_id[:, None],
      W1[:EMBED_DIM], W1[EMBED_DIM:],
      b1.reshape(1, 128), W2, b2.reshape(1, EMBED_DIM),
      W3.reshape(1, EMBED_DIM))
    return out


def kernel(user_id, anime_id, user_table, anime_table, W1, b1, W2, b2, W3, b3):
    batch = user_id.shape[0]
    n_user, n_anime = user_table.shape[0], anime_table.shape[0]
    rk = _make_relayout_kernel(n_user, n_anime)
    utab2, atab2 = rk(user_table.T, anime_table.T)
    gk = _make_gather_kernel(batch)
    upairs, apairs = gk((user_id // 2).reshape(-1, IDX_CHUNK),
                        (anime_id // 2).reshape(-1, IDX_CHUNK),
                        utab2, atab2)
    out = _mlp(upairs, apairs, user_id, anime_id,
               W1, b1, W2, b2, W3, block_b=2048)
    return out[:, None] + b3


# speedup vs baseline: 3.4480x; 3.4480x over previous
import functools

import jax
import jax.numpy as jnp
from jax import lax
from jax.experimental import pallas as pl
from jax.experimental.pallas import tpu as pltpu
from jax.experimental.pallas import tpu_sc as plsc

EMBED_DIM = 64
PAIR_DIM = 2 * EMBED_DIM
IDX_CHUNK = 128
NW = 32
PAIR_W = 2048


def _cdiv(a, b):
    return (a + b - 1) // b


def _pairize_body(t1_ref, t2_ref, out_ref):
    a = t1_ref[...].T
    b = t2_ref[...].T
    out_ref[...] = jnp.concatenate([a, b], axis=1)


def _pairize(tabT, n_rows):
    n_blocks = _cdiv(_cdiv(n_rows, 2), PAIR_W)
    half = n_blocks
    last = _cdiv(n_rows, PAIR_W) - 1
    out = pl.pallas_call(
        _pairize_body,
        grid=(n_blocks,),
        in_specs=[
            pl.BlockSpec((EMBED_DIM, PAIR_W), lambda i: (0, i)),
            pl.BlockSpec((EMBED_DIM, PAIR_W),
                         lambda i: (0, jnp.minimum(half + i, last))),
        ],
        out_specs=pl.BlockSpec((PAIR_W, PAIR_DIM), lambda i: (i, 0)),
        out_shape=jax.ShapeDtypeStruct((n_blocks * PAIR_W, PAIR_DIM),
                                       jnp.float32),
    )(tabT, tabT)
    return out, n_blocks * PAIR_W


def _make_gather_kernel(batch):
    chunks_per_worker = batch // (NW * IDX_CHUNK)
    rows_per_worker = chunks_per_worker * IDX_CHUNK
    mesh = plsc.VectorSubcoreMesh(core_axis_name="c", subcore_axis_name="s")

    @functools.partial(
        pl.kernel,
        out_type=(
            jax.ShapeDtypeStruct((batch, PAIR_DIM), jnp.float32),
            jax.ShapeDtypeStruct((batch, PAIR_DIM), jnp.float32),
        ),
        mesh=mesh,
        scratch_types=[
            pltpu.VMEM((batch // (NW * IDX_CHUNK), IDX_CHUNK), jnp.int32),
            pltpu.VMEM((batch // (NW * IDX_CHUNK), IDX_CHUNK), jnp.int32),
            pltpu.VMEM((batch // NW, PAIR_DIM), jnp.float32),
            pltpu.SemaphoreType.DMA,
        ],
    )
    def gather_kernel(uidx_hbm, aidx_hbm, utab2_hbm, atab2_hbm,
                      uout_hbm, aout_hbm,
                      uidx_v, aidx_v, rows_v, sem):
        wid = lax.axis_index("s") * 2 + lax.axis_index("c")
        crow = wid * chunks_per_worker
        base = wid * rows_per_worker
        pltpu.sync_copy(uidx_hbm.at[pl.ds(crow, chunks_per_worker)], uidx_v)
        pltpu.sync_copy(aidx_hbm.at[pl.ds(crow, chunks_per_worker)], aidx_v)
        copies = []
        for j in range(chunks_per_worker):
            copies.append(pltpu.async_copy(
                utab2_hbm.at[uidx_v.at[j]],
                rows_v.at[pl.ds(j * IDX_CHUNK, IDX_CHUNK)], sem))
        for c in copies:
            c.wait()
        pltpu.sync_copy(rows_v, uout_hbm.at[pl.ds(base, rows_per_worker)])
        copies = []
        for j in range(chunks_per_worker):
            copies.append(pltpu.async_copy(
                atab2_hbm.at[aidx_v.at[j]],
                rows_v.at[pl.ds(j * IDX_CHUNK, IDX_CHUNK)], sem))
        for c in copies:
            c.wait()
        pltpu.sync_copy(rows_v, aout_hbm.at[pl.ds(base, rows_per_worker)])

    return gather_kernel


def _make_mlp_body(h_user, h_anime):
    def _mlp_body(upair_ref, apair_ref, uid_ref, aid_ref,
                  w1u_ref, w1a_ref, b1_ref, w2_ref, b2_ref, w3_ref, out_ref):
        up = upair_ref[...]
        ap = apair_ref[...]
        usel = uid_ref[...] >= h_user
        asel = aid_ref[...] >= h_anime
        u = jnp.where(usel, up[:, EMBED_DIM:], up[:, :EMBED_DIM])
        a = jnp.where(asel, ap[:, EMBED_DIM:], ap[:, :EMBED_DIM])
        h1 = jnp.dot(u, w1u_ref[...], preferred_element_type=jnp.float32)
        h1 = h1 + jnp.dot(a, w1a_ref[...], preferred_element_type=jnp.float32)
        h1 = jnp.maximum(h1 + b1_ref[...], 0.0)
        h2 = jnp.dot(h1, w2_ref[...], preferred_element_type=jnp.float32)
        h2 = jnp.maximum(h2 + b2_ref[...], 0.0)
        out_ref[...] = jnp.sum(h2 * w3_ref[...], axis=1)
    return _mlp_body


def _mlp(upairs, apairs, user_id, anime_id, h_user, h_anime,
         W1, b1, W2, b2, W3, block_b):
    batch = upairs.shape[0]
    grid = (batch // block_b,)
    full = lambda i: (0, 0)
    out = pl.pallas_call(
        _make_mlp_body(h_user, h_anime),
        grid=grid,
        in_specs=[
            pl.BlockSpec((block_b, PAIR_DIM), lambda i: (i, 0)),
            pl.BlockSpec((block_b, PAIR_DIM), lambda i: (i, 0)),
            pl.BlockSpec((block_b, 1), lambda i: (i, 0)),
            pl.BlockSpec((block_b, 1), lambda i: (i, 0)),
            pl.BlockSpec((EMBED_DIM, 128), full),
            pl.BlockSpec((EMBED_DIM, 128), full),
            pl.BlockSpec((1, 128), full),
            pl.BlockSpec((128, EMBED_DIM), full),
            pl.BlockSpec((1, EMBED_DIM), full),
            pl.BlockSpec((1, EMBED_DIM), full),
        ],
        out_specs=pl.BlockSpec((block_b,), lambda i: (i,)),
        out_shape=jax.ShapeDtypeStruct((batch,), jnp.float32),
    )(upairs, apairs, user_id[:, None], anime_id[:, None],
      W1[:EMBED_DIM], W1[EMBED_DIM:],
      b1.reshape(1, 128), W2, b2.reshape(1, EMBED_DIM),
      W3.reshape(1, EMBED_DIM))
    return out


def kernel(user_id, anime_id, user_table, anime_table, W1, b1, W2, b2, W3, b3):
    batch = user_id.shape[0]
    utab2, h_user = _pairize(user_table.T, user_table.shape[0])
    atab2, h_anime = _pairize(anime_table.T, anime_table.shape[0])
    uidx = jnp.where(user_id < h_user, user_id, user_id - h_user)
    aidx = jnp.where(anime_id < h_anime, anime_id, anime_id - h_anime)
    gk = _make_gather_kernel(batch)
    upairs, apairs = gk(uidx.reshape(-1, IDX_CHUNK),
                        aidx.reshape(-1, IDX_CHUNK),
                        utab2, atab2)
    out = _mlp(upairs, apairs, user_id, anime_id, h_user, h_anime,
               W1, b1, W2, b2, W3, block_b=2048)
    return out[:, None] + b3


# speedup vs baseline: 4.1702x; 1.2095x over previous
import functools

import jax
import jax.numpy as jnp
from jax import lax
from jax.experimental import pallas as pl
from jax.experimental.pallas import tpu as pltpu
from jax.experimental.pallas import tpu_sc as plsc

EMBED_DIM = 64
PAIR_DIM = 2 * EMBED_DIM
IDX_CHUNK = 128
NW = 32
PAIR_W = 4096


def _cdiv(a, b):
    return (a + b - 1) // b


def _pairize_body(t1_ref, t2_ref, out_ref):
    a = t1_ref[...].T
    b = t2_ref[...].T
    out_ref[...] = jnp.concatenate([a, b], axis=1)


def _pairize(tabT, n_rows):
    n_blocks = _cdiv(_cdiv(n_rows, 2), PAIR_W)
    half = n_blocks
    last = _cdiv(n_rows, PAIR_W) - 1
    out = pl.pallas_call(
        _pairize_body,
        grid=(n_blocks,),
        in_specs=[
            pl.BlockSpec((EMBED_DIM, PAIR_W), lambda i: (0, i)),
            pl.BlockSpec((EMBED_DIM, PAIR_W),
                         lambda i: (0, jnp.minimum(half + i, last))),
        ],
        out_specs=pl.BlockSpec((PAIR_W, PAIR_DIM), lambda i: (i, 0)),
        out_shape=jax.ShapeDtypeStruct((n_blocks * PAIR_W, PAIR_DIM),
                                       jnp.float32),
    )(tabT, tabT)
    return out, n_blocks * PAIR_W


def _make_gather_kernel(batch):
    chunks_per_worker = batch // (NW * IDX_CHUNK)
    rows_per_worker = chunks_per_worker * IDX_CHUNK
    mesh = plsc.VectorSubcoreMesh(core_axis_name="c", subcore_axis_name="s")

    @functools.partial(
        pl.kernel,
        out_type=(
            jax.ShapeDtypeStruct((batch, PAIR_DIM), jnp.float32),
            jax.ShapeDtypeStruct((batch, PAIR_DIM), jnp.float32),
        ),
        mesh=mesh,
        scratch_types=[
            pltpu.VMEM((batch // (NW * IDX_CHUNK), IDX_CHUNK), jnp.int32),
            pltpu.VMEM((batch // (NW * IDX_CHUNK), IDX_CHUNK), jnp.int32),
            pltpu.VMEM((batch // NW, PAIR_DIM), jnp.float32),
            pltpu.SemaphoreType.DMA,
        ],
    )
    def gather_kernel(uidx_hbm, aidx_hbm, utab2_hbm, atab2_hbm,
                      uout_hbm, aout_hbm,
                      uidx_v, aidx_v, rows_v, sem):
        wid = lax.axis_index("s") * 2 + lax.axis_index("c")
        crow = wid * chunks_per_worker
        base = wid * rows_per_worker
        pltpu.sync_copy(uidx_hbm.at[pl.ds(crow, chunks_per_worker)], uidx_v)
        pltpu.sync_copy(aidx_hbm.at[pl.ds(crow, chunks_per_worker)], aidx_v)
        copies = []
        for j in range(chunks_per_worker):
            copies.append(pltpu.async_copy(
                utab2_hbm.at[uidx_v.at[j]],
                rows_v.at[pl.ds(j * IDX_CHUNK, IDX_CHUNK)], sem))
        for c in copies:
            c.wait()
        pltpu.sync_copy(rows_v, uout_hbm.at[pl.ds(base, rows_per_worker)])
        copies = []
        for j in range(chunks_per_worker):
            copies.append(pltpu.async_copy(
                atab2_hbm.at[aidx_v.at[j]],
                rows_v.at[pl.ds(j * IDX_CHUNK, IDX_CHUNK)], sem))
        for c in copies:
            c.wait()
        pltpu.sync_copy(rows_v, aout_hbm.at[pl.ds(base, rows_per_worker)])

    return gather_kernel


def _make_mlp_body(h_user, h_anime):
    def _mlp_body(upair_ref, apair_ref, uid_ref, aid_ref,
                  w1u_ref, w1a_ref, b1_ref, w2_ref, b2_ref, w3_ref, out_ref):
        up = upair_ref[...]
        ap = apair_ref[...]
        usel = uid_ref[...] >= h_user
        asel = aid_ref[...] >= h_anime
        u = jnp.where(usel, up[:, EMBED_DIM:], up[:, :EMBED_DIM])
        a = jnp.where(asel, ap[:, EMBED_DIM:], ap[:, :EMBED_DIM])
        h1 = jnp.dot(u, w1u_ref[...], preferred_element_type=jnp.float32)
        h1 = h1 + jnp.dot(a, w1a_ref[...], preferred_element_type=jnp.float32)
        h1 = jnp.maximum(h1 + b1_ref[...], 0.0)
        h2 = jnp.dot(h1, w2_ref[...], preferred_element_type=jnp.float32)
        h2 = jnp.maximum(h2 + b2_ref[...], 0.0)
        out_ref[...] = jnp.sum(h2 * w3_ref[...], axis=1)
    return _mlp_body


def _mlp(upairs, apairs, user_id, anime_id, h_user, h_anime,
         W1, b1, W2, b2, W3, block_b):
    batch = upairs.shape[0]
    grid = (batch // block_b,)
    full = lambda i: (0, 0)
    out = pl.pallas_call(
        _make_mlp_body(h_user, h_anime),
        grid=grid,
        in_specs=[
            pl.BlockSpec((block_b, PAIR_DIM), lambda i: (i, 0)),
            pl.BlockSpec((block_b, PAIR_DIM), lambda i: (i, 0)),
            pl.BlockSpec((block_b, 1), lambda i: (i, 0)),
            pl.BlockSpec((block_b, 1), lambda i: (i, 0)),
            pl.BlockSpec((EMBED_DIM, 128), full),
            pl.BlockSpec((EMBED_DIM, 128), full),
            pl.BlockSpec((1, 128), full),
            pl.BlockSpec((128, EMBED_DIM), full),
            pl.BlockSpec((1, EMBED_DIM), full),
            pl.BlockSpec((1, EMBED_DIM), full),
        ],
        out_specs=pl.BlockSpec((block_b,), lambda i: (i,)),
        out_shape=jax.ShapeDtypeStruct((batch,), jnp.float32),
    )(upairs, apairs, user_id[:, None], anime_id[:, None],
      W1[:EMBED_DIM], W1[EMBED_DIM:],
      b1.reshape(1, 128), W2, b2.reshape(1, EMBED_DIM),
      W3.reshape(1, EMBED_DIM))
    return out


def kernel(user_id, anime_id, user_table, anime_table, W1, b1, W2, b2, W3, b3):
    batch = user_id.shape[0]
    utab2, h_user = _pairize(user_table.T, user_table.shape[0])
    atab2, h_anime = _pairize(anime_table.T, anime_table.shape[0])
    uidx = jnp.where(user_id < h_user, user_id, user_id - h_user)
    aidx = jnp.where(anime_id < h_anime, anime_id, anime_id - h_anime)
    gk = _make_gather_kernel(batch)
    upairs, apairs = gk(uidx.reshape(-1, IDX_CHUNK),
                        aidx.reshape(-1, IDX_CHUNK),
                        utab2, atab2)
    out = _mlp(upairs, apairs, user_id, anime_id, h_user, h_anime,
               W1, b1, W2, b2, W3, block_b=2048)
    return out[:, None] + b3


# speedup vs baseline: 4.6187x; 1.1075x over previous
import functools

import jax
import jax.numpy as jnp
from jax import lax
from jax.experimental import pallas as pl
from jax.experimental.pallas import tpu as pltpu
from jax.experimental.pallas import tpu_sc as plsc

EMBED_DIM = 64
PAIR_DIM = 2 * EMBED_DIM
IDX_CHUNK = 128
NW = 32
PAIR_W = 8192


def _cdiv(a, b):
    return (a + b - 1) // b


def _pairize_body(t1_ref, t2_ref, out_ref):
    a = t1_ref[...].T
    b = t2_ref[...].T
    out_ref[...] = jnp.concatenate([a, b], axis=1)


def _pairize(tabT, n_rows):
    n_blocks = _cdiv(_cdiv(n_rows, 2), PAIR_W)
    half = n_blocks
    last = _cdiv(n_rows, PAIR_W) - 1
    out = pl.pallas_call(
        _pairize_body,
        grid=(n_blocks,),
        in_specs=[
            pl.BlockSpec((EMBED_DIM, PAIR_W), lambda i: (0, i)),
            pl.BlockSpec((EMBED_DIM, PAIR_W),
                         lambda i: (0, jnp.minimum(half + i, last))),
        ],
        out_specs=pl.BlockSpec((PAIR_W, PAIR_DIM), lambda i: (i, 0)),
        out_shape=jax.ShapeDtypeStruct((n_blocks * PAIR_W, PAIR_DIM),
                                       jnp.float32),
    )(tabT, tabT)
    return out, n_blocks * PAIR_W


def _make_gather_kernel(batch):
    chunks_per_worker = batch // (NW * IDX_CHUNK)
    rows_per_worker = chunks_per_worker * IDX_CHUNK
    mesh = plsc.VectorSubcoreMesh(core_axis_name="c", subcore_axis_name="s")

    @functools.partial(
        pl.kernel,
        out_type=(
            jax.ShapeDtypeStruct((batch, PAIR_DIM), jnp.float32),
            jax.ShapeDtypeStruct((batch, PAIR_DIM), jnp.float32),
        ),
        mesh=mesh,
        scratch_types=[
            pltpu.VMEM((batch // (NW * IDX_CHUNK), IDX_CHUNK), jnp.int32),
            pltpu.VMEM((batch // (NW * IDX_CHUNK), IDX_CHUNK), jnp.int32),
            pltpu.VMEM((batch // NW, PAIR_DIM), jnp.float32),
            pltpu.SemaphoreType.DMA,
        ],
    )
    def gather_kernel(uidx_hbm, aidx_hbm, utab2_hbm, atab2_hbm,
                      uout_hbm, aout_hbm,
                      uidx_v, aidx_v, rows_v, sem):
        wid = lax.axis_index("s") * 2 + lax.axis_index("c")
        crow = wid * chunks_per_worker
        base = wid * rows_per_worker
        pltpu.sync_copy(uidx_hbm.at[pl.ds(crow, chunks_per_worker)], uidx_v)
        pltpu.sync_copy(aidx_hbm.at[pl.ds(crow, chunks_per_worker)], aidx_v)
        copies = []
        for j in range(chunks_per_worker):
            copies.append(pltpu.async_copy(
                utab2_hbm.at[uidx_v.at[j]],
                rows_v.at[pl.ds(j * IDX_CHUNK, IDX_CHUNK)], sem))
        for c in copies:
            c.wait()
        pltpu.sync_copy(rows_v, uout_hbm.at[pl.ds(base, rows_per_worker)])
        copies = []
        for j in range(chunks_per_worker):
            copies.append(pltpu.async_copy(
                atab2_hbm.at[aidx_v.at[j]],
                rows_v.at[pl.ds(j * IDX_CHUNK, IDX_CHUNK)], sem))
        for c in copies:
            c.wait()
        pltpu.sync_copy(rows_v, aout_hbm.at[pl.ds(base, rows_per_worker)])

    return gather_kernel


def _make_mlp_body(h_user, h_anime):
    def _mlp_body(upair_ref, apair_ref, uid_ref, aid_ref,
                  w1u_ref, w1a_ref, b1_ref, w2_ref, b2_ref, w3_ref, out_ref):
        up = upair_ref[...]
        ap = apair_ref[...]
        usel = uid_ref[...] >= h_user
        asel = aid_ref[...] >= h_anime
        u = jnp.where(usel, up[:, EMBED_DIM:], up[:, :EMBED_DIM])
        a = jnp.where(asel, ap[:, EMBED_DIM:], ap[:, :EMBED_DIM])
        h1 = jnp.dot(u, w1u_ref[...], preferred_element_type=jnp.float32)
        h1 = h1 + jnp.dot(a, w1a_ref[...], preferred_element_type=jnp.float32)
        h1 = jnp.maximum(h1 + b1_ref[...], 0.0)
        h2 = jnp.dot(h1, w2_ref[...], preferred_element_type=jnp.float32)
        h2 = jnp.maximum(h2 + b2_ref[...], 0.0)
        out_ref[...] = jnp.sum(h2 * w3_ref[...], axis=1)
    return _mlp_body


def _mlp(upairs, apairs, user_id, anime_id, h_user, h_anime,
         W1, b1, W2, b2, W3, block_b):
    batch = upairs.shape[0]
    grid = (batch // block_b,)
    full = lambda i: (0, 0)
    out = pl.pallas_call(
        _make_mlp_body(h_user, h_anime),
        grid=grid,
        in_specs=[
            pl.BlockSpec((block_b, PAIR_DIM), lambda i: (i, 0)),
            pl.BlockSpec((block_b, PAIR_DIM), lambda i: (i, 0)),
            pl.BlockSpec((block_b, 1), lambda i: (i, 0)),
            pl.BlockSpec((block_b, 1), lambda i: (i, 0)),
            pl.BlockSpec((EMBED_DIM, 128), full),
            pl.BlockSpec((EMBED_DIM, 128), full),
            pl.BlockSpec((1, 128), full),
            pl.BlockSpec((128, EMBED_DIM), full),
            pl.BlockSpec((1, EMBED_DIM), full),
            pl.BlockSpec((1, EMBED_DIM), full),
        ],
        out_specs=pl.BlockSpec((block_b,), lambda i: (i,)),
        out_shape=jax.ShapeDtypeStruct((batch,), jnp.float32),
    )(upairs, apairs, user_id[:, None], anime_id[:, None],
      W1[:EMBED_DIM], W1[EMBED_DIM:],
      b1.reshape(1, 128), W2, b2.reshape(1, EMBED_DIM),
      W3.reshape(1, EMBED_DIM))
    return out


def kernel(user_id, anime_id, user_table, anime_table, W1, b1, W2, b2, W3, b3):
    batch = user_id.shape[0]
    utab2, h_user = _pairize(user_table.T, user_table.shape[0])
    atab2, h_anime = _pairize(anime_table.T, anime_table.shape[0])
    uidx = jnp.where(user_id < h_user, user_id, user_id - h_user)
    aidx = jnp.where(anime_id < h_anime, anime_id, anime_id - h_anime)
    gk = _make_gather_kernel(batch)
    upairs, apairs = gk(uidx.reshape(-1, IDX_CHUNK),
                        aidx.reshape(-1, IDX_CHUNK),
                        utab2, atab2)
    out = _mlp(upairs, apairs, user_id, anime_id, h_user, h_anime,
               W1, b1, W2, b2, W3, block_b=2048)
    return out[:, None] + b3
